# baseline (speedup 1.0000x reference)
import jax
import jax.numpy as jnp
from jax import lax
from jax.experimental import pallas as pl
from jax.experimental.pallas import tpu as pltpu
from jax.experimental.pallas import tpu_sc as plsc

NC = 2
NS = 16
LANES = 16
NW = NC * NS

B, L, V, D = 4096, 200, 1000000, 64
DP = 128
N = B * L
N_PER_W = N // NW
CHUNK = 128
NBUF = 4
N_CHUNKS = N_PER_W // CHUNK
LOOKAHEAD = 3
STEADY = (N_CHUNKS - 1 - LOOKAHEAD) // NBUF


def _body(x_hbm, tok_hbm, pos_hbm, out_hbm, *refs):
  idx_all = refs[0]
  tok = refs[1:1 + NBUF]
  pos_l = refs[1 + NBUF]
  gsem = refs[2 + NBUF:2 + 2 * NBUF]
  wsem = refs[2 + 2 * NBUF:2 + 3 * NBUF]

  wid = lax.axis_index("s") * NC + lax.axis_index("c")
  w_base = wid * N_PER_W

  pltpu.sync_copy(pos_hbm, pos_l)
  pltpu.sync_copy(x_hbm.at[pl.ds(w_base, N_PER_W)], idx_all)

  def fire_gather(c, k):
    pltpu.async_copy(tok_hbm.at[idx_all.at[pl.ds(c * CHUNK, CHUNK)]], tok[k],
                     gsem[k])

  def wait_gather(c, k):
    pltpu.make_async_copy(tok_hbm.at[idx_all.at[pl.ds(c * CHUNK, CHUNK)]],
                          tok[k], gsem[k]).wait()

  def wait_writeback(c, k):
    pltpu.make_async_copy(tok[k], out_hbm.at[pl.ds(w_base + c * CHUNK, CHUNK)],
                          wsem[k]).wait()

  def combine(c, k):
    base = w_base + c * CHUNK

    def add_body(g, _):
      xv = idx_all[pl.ds(c * CHUNK + g * LANES, LANES)]
      t = base + g * LANES + lax.iota(jnp.int32, LANES)
      pv = jnp.where(xv == 0, jnp.zeros((LANES,), jnp.int32),
                     lax.rem(t, L) + 1)
      for kk in range(LANES):
        r = g * LANES + kk
        p = pv[kk]
        for j in range(D // LANES):
          s = pl.ds(j * LANES, LANES)
          tok[k][r, s] = tok[k][r, s] + pos_l[p, s]
      return 0
    lax.fori_loop(0, CHUNK // LANES, add_body, 0)

    pltpu.async_copy(tok[k], out_hbm.at[pl.ds(base, CHUNK)], wsem[k])

  for c in range(LOOKAHEAD):
    fire_gather(c, c % NBUF)
  wait_gather(0, 0)
  combine(0, 0)
  fire_gather(LOOKAHEAD, LOOKAHEAD % NBUF)

  def macro_body(i, _):
    c0 = 1 + i * NBUF
    for k in range(NBUF):
      c = c0 + k
      bc = (1 + k) % NBUF
      wait_gather(c, bc)
      combine(c, bc)
      bp = (1 + k + LOOKAHEAD) % NBUF
      wait_writeback(c - 1, bp)
      fire_gather(c + LOOKAHEAD, bp)
    return 0
  lax.fori_loop(0, STEADY, macro_body, 0)

  for c in range(1 + STEADY * NBUF, N_CHUNKS):
    wait_gather(c, c % NBUF)
    combine(c, c % NBUF)

  for c in range(N_CHUNKS - NBUF, N_CHUNKS):
    wait_writeback(c, c % NBUF)


VB = 128
NFULL = V // VB
NREG = NFULL - (NFULL % NW)


def _tbody(tin_hbm, tail_hbm, out_hbm, inb0, inb1, outb0, outb1, rsem0, rsem1,
           wsem0, wsem1):
  wid = lax.axis_index("s") * NC + lax.axis_index("c")
  inb = (inb0, inb1)
  outb = (outb0, outb1)
  rsem = (rsem0, rsem1)
  wsem = (wsem0, wsem1)

  nblk = NREG // NW

  def col0(i):
    return (wid + i * NW) * VB

  def fire_read(i, b):
    pltpu.async_copy(tin_hbm.at[:, pl.ds(col0(i), VB)], inb[b], rsem[b])

  def wait_read(i, b):
    pltpu.make_async_copy(tin_hbm.at[:, pl.ds(col0(i), VB)], inb[b],
                          rsem[b]).wait()

  def wait_write(i, b):
    pltpu.make_async_copy(outb[b], out_hbm.at[pl.ds(col0(i) * DP, VB * DP)],
                          wsem[b]).wait()

  def transpose(b):
    def tgroup(t0, _):
      base = t0 * DP + lax.iota(jnp.int32, LANES) * DP
      for d in range(D):
        vals = inb[b][d, pl.ds(t0, LANES)]
        plsc.store_scatter(outb[b], [base + d], vals)
      return 0
    lax.fori_loop(0, VB // LANES, lambda g, _: tgroup(g * LANES, _), 0,
                  unroll=False)

  def emit(i, b):
    transpose(b)
    pltpu.async_copy(outb[b], out_hbm.at[pl.ds(col0(i) * DP, VB * DP)],
                     wsem[b])

  fire_read(0, 0)
  fire_read(1, 1)
  wait_read(0, 0)
  emit(0, 0)
  fire_read(2, 0)
  wait_read(1, 1)
  emit(1, 1)
  fire_read(3, 1)

  def pair_body(m, _):
    i0 = 2 * m
    wait_read(i0, 0)
    wait_write(i0 - 2, 0)
    emit(i0, 0)
    fire_read(i0 + 2, 0)
    wait_read(i0 + 1, 1)
    wait_write(i0 - 1, 1)
    emit(i0 + 1, 1)
    fire_read(i0 + 3, 1)
    return 0
  lax.fori_loop(1, nblk // 2 - 1, pair_body, 0)

  wait_read(nblk - 2, 0)
  wait_write(nblk - 4, 0)
  emit(nblk - 2, 0)
  wait_read(nblk - 1, 1)
  wait_write(nblk - 3, 1)
  emit(nblk - 1, 1)
  wait_write(nblk - 2, 0)
  wait_write(nblk - 1, 1)

  for e in range(NFULL - NREG):
    @pl.when(wid == e)
    def _(e=e):
      c0 = (NREG + e) * VB
      pltpu.async_copy(tin_hbm.at[:, pl.ds(c0, VB)], inb[0], rsem[0])
      pltpu.make_async_copy(tin_hbm.at[:, pl.ds(c0, VB)], inb[0],
                            rsem[0]).wait()
      transpose(0)
      pltpu.sync_copy(outb[0], out_hbm.at[pl.ds(c0 * DP, VB * DP)])

  @pl.when(wid == NFULL - NREG)
  def _():
    pltpu.async_copy(tail_hbm, inb[0], rsem[0])
    pltpu.make_async_copy(tail_hbm, inb[0], rsem[0]).wait()

    def tgroup(t0, _):
      base = t0 * DP + lax.iota(jnp.int32, LANES) * DP
      for d in range(D):
        vals = inb[0][d, pl.ds(t0, LANES)]
        plsc.store_scatter(outb[0], [base + d], vals)
      return 0
    lax.fori_loop(0, (V - NFULL * VB) // LANES,
                  lambda g, _: tgroup(g * LANES, _), 0, unroll=False)
    pltpu.sync_copy(outb[0].at[pl.ds(0, (V - NFULL * VB) * DP)],
                    out_hbm.at[pl.ds(NFULL * VB * DP, (V - NFULL * VB) * DP)])


def _transpose_pad(token_table):
  tfn = pl.kernel(
      _tbody,
      out_type=jax.ShapeDtypeStruct((V * DP,), jnp.float32),
      mesh=plsc.VectorSubcoreMesh(core_axis_name="c", subcore_axis_name="s"),
      scratch_types=[
          pltpu.VMEM((D, VB), jnp.float32),
          pltpu.VMEM((D, VB), jnp.float32),
          pltpu.VMEM((VB * DP,), jnp.float32),
          pltpu.VMEM((VB * DP,), jnp.float32),
          pltpu.SemaphoreType.DMA,
          pltpu.SemaphoreType.DMA,
          pltpu.SemaphoreType.DMA,
          pltpu.SemaphoreType.DMA,
      ],
      compiler_params=pltpu.CompilerParams(needs_layout_passes=False),
  )
  tin = token_table.T
  tail = jnp.pad(lax.slice(tin, (0, NFULL * VB), (D, V)),
                 ((0, 0), (0, VB - (V - NFULL * VB))))
  return tfn(tin, tail).reshape(V, DP)


@jax.jit
def kernel(x, token_table, pos_table):
  scratch = (
      [pltpu.VMEM((N_PER_W,), jnp.int32)]
      + [pltpu.VMEM((CHUNK, DP), jnp.float32) for _ in range(NBUF)]
      + [pltpu.VMEM((L + 1, D), jnp.float32)]
      + [pltpu.SemaphoreType.DMA for _ in range(2 * NBUF)]
  )
  kfn = pl.kernel(
      _body,
      out_type=jax.ShapeDtypeStruct((N, DP), jnp.float32),
      mesh=plsc.VectorSubcoreMesh(core_axis_name="c", subcore_axis_name="s"),
      scratch_types=scratch,
  )
  tt = _transpose_pad(token_table)
  out = kfn(x.reshape(N), tt, pos_table)
  return out[:, :D].reshape(B, L, D)


# speedup vs baseline: 1.5815x; 1.5815x over previous
import jax
import jax.numpy as jnp
from jax import lax
from jax.experimental import pallas as pl
from jax.experimental.pallas import tpu as pltpu
from jax.experimental.pallas import tpu_sc as plsc

NC = 2
NS = 16
LANES = 16
NW = NC * NS

B, L, V, D = 4096, 200, 1000000, 64
DP = 128
N = B * L
N_PER_W = N // NW
CHUNK = 128
NBUF = 4
N_CHUNKS = N_PER_W // CHUNK
LOOKAHEAD = 3
STEADY = (N_CHUNKS - 1 - LOOKAHEAD) // NBUF


def _body(x_hbm, tok_hbm, pos_hbm, out_hbm, *refs):
  idx_all = refs[0]
  tok = refs[1:1 + NBUF]
  pos_l = refs[1 + NBUF]
  gsem = refs[2 + NBUF:2 + 2 * NBUF]
  wsem = refs[2 + 2 * NBUF:2 + 3 * NBUF]

  wid = lax.axis_index("s") * NC + lax.axis_index("c")
  w_base = wid * N_PER_W

  pltpu.sync_copy(pos_hbm, pos_l.at[pl.ds(0, L + 1)])
  pltpu.sync_copy(x_hbm.at[pl.ds(w_base, N_PER_W)], idx_all)
  for r in range(LANES - 1):
    for j in range(D // LANES):
      s = pl.ds(j * LANES, LANES)
      pos_l[L + 1 + r, s] = pos_l[r + 1, s]

  def fire_gather(c, k):
    pltpu.async_copy(tok_hbm.at[idx_all.at[pl.ds(c * CHUNK, CHUNK)]], tok[k],
                     gsem[k])

  def wait_gather(c, k):
    pltpu.make_async_copy(tok_hbm.at[idx_all.at[pl.ds(c * CHUNK, CHUNK)]],
                          tok[k], gsem[k]).wait()

  def wait_writeback(c, k):
    pltpu.make_async_copy(tok[k], out_hbm.at[pl.ds(w_base + c * CHUNK, CHUNK)],
                          wsem[k]).wait()

  def combine(c, k):
    base = w_base + c * CHUNK

    def add_body(g, _):
      xv = idx_all[pl.ds(c * CHUNK + g * LANES, LANES)]
      l0 = lax.rem(base + g * LANES, L)
      for kk in range(LANES):
        r = g * LANES + kk
        for j in range(D // LANES):
          s = pl.ds(j * LANES, LANES)
          tok[k][r, s] = tok[k][r, s] + pos_l[l0 + 1 + kk, s]

      @pl.when(jnp.any(xv == 0))
      def _():
        for kk in range(LANES):
          r = g * LANES + kk

          @pl.when(xv[kk] == 0)
          def _(r=r, kk=kk):
            for j in range(D // LANES):
              s = pl.ds(j * LANES, LANES)
              tok[k][r, s] = (tok[k][r, s] + pos_l[0, s]
                              - pos_l[l0 + 1 + kk, s])
      return 0
    lax.fori_loop(0, CHUNK // LANES, add_body, 0)

    pltpu.async_copy(tok[k], out_hbm.at[pl.ds(base, CHUNK)], wsem[k])

  for c in range(LOOKAHEAD):
    fire_gather(c, c % NBUF)
  wait_gather(0, 0)
  combine(0, 0)
  fire_gather(LOOKAHEAD, LOOKAHEAD % NBUF)

  def macro_body(i, _):
    c0 = 1 + i * NBUF
    for k in range(NBUF):
      c = c0 + k
      bc = (1 + k) % NBUF
      wait_gather(c, bc)
      combine(c, bc)
      bp = (1 + k + LOOKAHEAD) % NBUF
      wait_writeback(c - 1, bp)
      fire_gather(c + LOOKAHEAD, bp)
    return 0
  lax.fori_loop(0, STEADY, macro_body, 0)

  for c in range(1 + STEADY * NBUF, N_CHUNKS):
    wait_gather(c, c % NBUF)
    combine(c, c % NBUF)

  for c in range(N_CHUNKS - NBUF, N_CHUNKS):
    wait_writeback(c, c % NBUF)


@jax.jit
def kernel(x, token_table, pos_table):
  scratch = (
      [pltpu.VMEM((N_PER_W,), jnp.int32)]
      + [pltpu.VMEM((CHUNK, DP), jnp.float32) for _ in range(NBUF)]
      + [pltpu.VMEM((L + LANES, D), jnp.float32)]
      + [pltpu.SemaphoreType.DMA for _ in range(2 * NBUF)]
  )
  kfn = pl.kernel(
      _body,
      out_type=jax.ShapeDtypeStruct((N, DP), jnp.float32),
      mesh=plsc.VectorSubcoreMesh(core_axis_name="c", subcore_axis_name="s"),
      scratch_types=scratch,
      compiler_params=pltpu.CompilerParams(needs_layout_passes=False),
  )
  tt = jnp.pad(token_table, ((0, 0), (0, DP - D)))
  out = kfn(x.reshape(N), tt, pos_table)
  return out[:, :D].reshape(B, L, D)


# speedup vs baseline: 1.8738x; 1.1849x over previous
import jax
import jax.numpy as jnp
from jax import lax
from jax.experimental import pallas as pl
from jax.experimental.pallas import tpu as pltpu
from jax.experimental.pallas import tpu_sc as plsc

NC = 2
NS = 16
LANES = 16
NW = NC * NS

B, L, V, D = 4096, 200, 1000000, 64
DP = 128
N = B * L
N_PER_W = N // NW
CHUNK = 128
NBUF = 4
N_CHUNKS = N_PER_W // CHUNK
LOOKAHEAD = 3
STEADY = (N_CHUNKS - 1 - LOOKAHEAD) // NBUF


def _body(x_hbm, tok_hbm, pos_hbm, out_hbm, *refs):
  idx_all = refs[0]
  tok = refs[1:1 + NBUF]
  pos_l = refs[1 + NBUF]
  gsem = refs[2 + NBUF:2 + 2 * NBUF]
  wsem = refs[2 + 2 * NBUF:2 + 3 * NBUF]
  psem = refs[2 + 3 * NBUF:2 + 4 * NBUF]
  pos_sh = refs[2 + 4 * NBUF]

  wid = lax.axis_index("s") * NC + lax.axis_index("c")
  w_base = wid * N_PER_W

  pltpu.sync_copy(pos_hbm, pos_l.at[pl.ds(0, L + 1)])
  pltpu.sync_copy(x_hbm.at[pl.ds(w_base, N_PER_W)], idx_all)
  for r in range(LANES - 1):
    for j in range(D // LANES):
      s = pl.ds(j * LANES, LANES)
      pos_l[L + 1 + r, s] = pos_l[r + 1, s]

  @pl.when(lax.axis_index("s") == 0)
  def _():
    pltpu.sync_copy(pos_hbm, pos_sh.at[pl.ds(0, L + 1)])
    for r in range(LANES - 1):
      pltpu.sync_copy(pos_hbm.at[pl.ds(r + 1, 1)],
                      pos_sh.at[pl.ds(L + 1 + r, 1)])
  plsc.subcore_barrier()

  def fire_gather(c, k):
    base = w_base + c * CHUNK
    for g in range(CHUNK // LANES):
      l0 = lax.rem(base + g * LANES, L)
      pltpu.async_copy(pos_sh.at[pl.ds(l0 + 1, LANES)],
                       tok[k].at[pl.ds(g * LANES, LANES)], psem[k])
    for g in range(CHUNK // LANES):
      l0 = lax.rem(base + g * LANES, L)
      pltpu.make_async_copy(pos_sh.at[pl.ds(l0 + 1, LANES)],
                            tok[k].at[pl.ds(g * LANES, LANES)],
                            psem[k]).wait()
    pltpu.async_copy(tok_hbm.at[idx_all.at[pl.ds(c * CHUNK, CHUNK)]], tok[k],
                     gsem[k], add=True)

  def wait_gather(c, k):
    pltpu.make_async_copy(tok_hbm.at[idx_all.at[pl.ds(c * CHUNK, CHUNK)]],
                          tok[k], gsem[k]).wait()

  def wait_writeback(c, k):
    pltpu.make_async_copy(tok[k], out_hbm.at[pl.ds(w_base + c * CHUNK, CHUNK)],
                          wsem[k]).wait()

  def combine(c, k):
    base = w_base + c * CHUNK

    def add_body(g, _):
      xv = idx_all[pl.ds(c * CHUNK + g * LANES, LANES)]
      l0 = lax.rem(base + g * LANES, L)
      @pl.when(jnp.any(xv == 0))
      def _():
        for kk in range(LANES):
          r = g * LANES + kk

          @pl.when(xv[kk] == 0)
          def _(r=r, kk=kk):
            for j in range(D // LANES):
              s = pl.ds(j * LANES, LANES)
              tok[k][r, s] = (tok[k][r, s] + pos_l[0, s]
                              - pos_l[l0 + 1 + kk, s])
      return 0
    lax.fori_loop(0, CHUNK // LANES, add_body, 0)

    pltpu.async_copy(tok[k], out_hbm.at[pl.ds(base, CHUNK)], wsem[k])

  for c in range(LOOKAHEAD):
    fire_gather(c, c % NBUF)
  wait_gather(0, 0)
  combine(0, 0)
  fire_gather(LOOKAHEAD, LOOKAHEAD % NBUF)

  def macro_body(i, _):
    c0 = 1 + i * NBUF
    for k in range(NBUF):
      c = c0 + k
      bc = (1 + k) % NBUF
      wait_gather(c, bc)
      combine(c, bc)
      bp = (1 + k + LOOKAHEAD) % NBUF
      wait_writeback(c - 1, bp)
      fire_gather(c + LOOKAHEAD, bp)
    return 0
  lax.fori_loop(0, STEADY, macro_body, 0)

  for c in range(1 + STEADY * NBUF, N_CHUNKS):
    wait_gather(c, c % NBUF)
    combine(c, c % NBUF)

  for c in range(N_CHUNKS - NBUF, N_CHUNKS):
    wait_writeback(c, c % NBUF)


@jax.jit
def kernel(x, token_table, pos_table):
  scratch = (
      [pltpu.VMEM((N_PER_W,), jnp.int32)]
      + [pltpu.VMEM((CHUNK, DP), jnp.float32) for _ in range(NBUF)]
      + [pltpu.VMEM((L + LANES, DP), jnp.float32)]
      + [pltpu.SemaphoreType.DMA for _ in range(3 * NBUF)]
      + [pltpu.VMEM_SHARED((L + LANES, DP), jnp.float32)]
  )
  kfn = pl.kernel(
      _body,
      out_type=jax.ShapeDtypeStruct((N, DP), jnp.float32),
      mesh=plsc.VectorSubcoreMesh(core_axis_name="c", subcore_axis_name="s"),
      scratch_types=scratch,
      compiler_params=pltpu.CompilerParams(needs_layout_passes=False),
  )
  tt = jnp.pad(token_table, ((0, 0), (0, DP - D)))
  pos128 = jnp.pad(pos_table, ((0, 0), (0, DP - D)))
  out = kfn(x.reshape(N), tt, pos128)
  return out[:, :D].reshape(B, L, D)
